# Initial kernel scaffold; baseline (speedup 1.0000x reference)
import jax
import jax.numpy as jnp
from jax.experimental import pallas as pl


def kernel(inputs, supports, batch_seen, enc0_ru_W, enc0_ru_b, enc0_c_W, enc0_c_b, enc1_ru_W, enc1_ru_b, enc1_c_W, enc1_c_b, dec0_ru_W, dec0_ru_b, dec0_c_W, dec0_c_b, dec1_ru_W, dec1_ru_b, dec1_c_W, dec1_c_b, dec_out_W, dec_out_b):
    raise NotImplementedError("write your pallas kernel here")


# speedup vs baseline: 5.5683x; 5.5683x over previous
import jax
import jax.numpy as jnp
from jax.experimental import pallas as pl
from jax.experimental.pallas import tpu as pltpu

B = 16
T = 12
N = 512
IN_DIM = 2
OUT_DIM = 1
HID = 64
N_SUP = 2
K_HOP = 2
N_PRED = 12
BN = B * N


def _hop(v, At):
    parts = [
        jnp.dot(v[:, b * N:(b + 1) * N], At, preferred_element_type=jnp.float32)
        for b in range(B)
    ]
    return jnp.concatenate(parts, axis=1)


def _gconv(x, st, A0, A1, Wt, bias):
    cat = jnp.concatenate([x, st], axis=0)
    feats = [cat]
    for A in (A0, A1):
        v = cat
        for _ in range(K_HOP):
            v = _hop(v, A)
            feats.append(v)
    h = jnp.concatenate(feats, axis=0)
    return jnp.dot(Wt, h, preferred_element_type=jnp.float32) + bias


def _sigmoid(x):
    return 1.0 / (1.0 + jnp.exp(-x))


def _cell(x, st, A0, A1, ruWt, rub, cWt, cb):
    ru = _sigmoid(_gconv(x, st, A0, A1, ruWt, rub))
    r = ru[:HID]
    u = ru[HID:]
    c = jnp.tanh(_gconv(x, r * st, A0, A1, cWt, cb))
    return u * st + (1.0 - u) * c


def _dcrnn_kernel(xin_ref, A0_ref, A1_ref,
                  e0ruW_ref, e0rub_ref, e0cW_ref, e0cb_ref,
                  e1ruW_ref, e1rub_ref, e1cW_ref, e1cb_ref,
                  d0ruW_ref, d0rub_ref, d0cW_ref, d0cb_ref,
                  d1ruW_ref, d1rub_ref, d1cW_ref, d1cb_ref,
                  doW_ref, dob_ref,
                  out_ref,
                  st0_ref, st1_ref, xd_ref):
    A0 = A0_ref[...]
    A1 = A1_ref[...]
    e0 = (e0ruW_ref[...], e0rub_ref[...], e0cW_ref[...], e0cb_ref[...])
    e1 = (e1ruW_ref[...], e1rub_ref[...], e1cW_ref[...], e1cb_ref[...])
    d0 = (d0ruW_ref[...], d0rub_ref[...], d0cW_ref[...], d0cb_ref[...])
    d1 = (d1ruW_ref[...], d1rub_ref[...], d1cW_ref[...], d1cb_ref[...])

    st0_ref[...] = jnp.zeros((HID, BN), jnp.float32)
    st1_ref[...] = jnp.zeros((HID, BN), jnp.float32)

    def enc_body(t, carry):
        x = xin_ref[t]
        s0 = _cell(x, st0_ref[...], A0, A1, *e0)
        st0_ref[...] = s0
        s1 = _cell(s0, st1_ref[...], A0, A1, *e1)
        st1_ref[...] = s1
        return carry

    jax.lax.fori_loop(0, T, enc_body, 0)

    xd_ref[...] = jnp.zeros((8, BN), jnp.float32)

    def dec_body(t, carry):
        x = xd_ref[...]
        s0 = _cell(x, st0_ref[...], A0, A1, *d0)
        st0_ref[...] = s0
        s1 = _cell(s0, st1_ref[...], A0, A1, *d1)
        st1_ref[...] = s1
        p = jnp.dot(doW_ref[...], s1, preferred_element_type=jnp.float32) + dob_ref[...]
        out_ref[t] = p
        xd_ref[...] = p
        return carry

    jax.lax.fori_loop(0, N_PRED, dec_body, 0)


def kernel(inputs, supports, batch_seen,
           enc0_ru_W, enc0_ru_b, enc0_c_W, enc0_c_b,
           enc1_ru_W, enc1_ru_b, enc1_c_W, enc1_c_b,
           dec0_ru_W, dec0_ru_b, dec0_c_W, dec0_c_b,
           dec1_ru_W, dec1_ru_b, dec1_c_W, dec1_c_b,
           dec_out_W, dec_out_b):
    f32 = jnp.float32

    xin = jnp.transpose(inputs, (1, 3, 0, 2)).reshape(T, IN_DIM, BN).astype(f32)

    A0 = jnp.transpose(supports[0]).astype(f32)
    A1 = jnp.transpose(supports[1]).astype(f32)

    def prep(W, b, dx, pad_x):
        din = dx + HID
        dout = W.shape[1]
        M = W.shape[0] // din
        Wt = jnp.transpose(W)
        if pad_x != dx:
            blocks = []
            for m in range(M):
                blk = Wt[:, m * din:(m + 1) * din]
                blocks.append(jnp.concatenate(
                    [blk[:, :dx], jnp.zeros((dout, pad_x - dx), f32), blk[:, dx:]],
                    axis=1))
            Wt = jnp.concatenate(blocks, axis=1)
        return Wt.astype(f32), b.reshape(-1, 1).astype(f32)

    e0ruW, e0rub = prep(enc0_ru_W, enc0_ru_b, IN_DIM, IN_DIM)
    e0cW, e0cb = prep(enc0_c_W, enc0_c_b, IN_DIM, IN_DIM)
    e1ruW, e1rub = prep(enc1_ru_W, enc1_ru_b, HID, HID)
    e1cW, e1cb = prep(enc1_c_W, enc1_c_b, HID, HID)
    d0ruW, d0rub = prep(dec0_ru_W, dec0_ru_b, OUT_DIM, 8)
    d0cW, d0cb = prep(dec0_c_W, dec0_c_b, OUT_DIM, 8)
    d1ruW, d1rub = prep(dec1_ru_W, dec1_ru_b, HID, HID)
    d1cW, d1cb = prep(dec1_c_W, dec1_c_b, HID, HID)

    doW = jnp.concatenate(
        [jnp.transpose(dec_out_W), jnp.zeros((8 - OUT_DIM, HID), f32)], axis=0)
    dob = jnp.concatenate(
        [dec_out_b.reshape(OUT_DIM, 1), jnp.zeros((8 - OUT_DIM, 1), f32)], axis=0)

    out = pl.pallas_call(
        _dcrnn_kernel,
        out_shape=jax.ShapeDtypeStruct((N_PRED, 8, BN), f32),
        scratch_shapes=[
            pltpu.VMEM((HID, BN), f32),
            pltpu.VMEM((HID, BN), f32),
            pltpu.VMEM((8, BN), f32),
        ],
    )(xin, A0, A1,
      e0ruW, e0rub, e0cW, e0cb,
      e1ruW, e1rub, e1cW, e1cb,
      d0ruW, d0rub, d0cW, d0cb,
      d1ruW, d1rub, d1cW, d1cb,
      doW, dob)

    preds = out[:, 0, :].reshape(N_PRED, B, N)
    return jnp.transpose(preds, (1, 0, 2))[..., None]
